# Initial kernel scaffold; baseline (speedup 1.0000x reference)
import functools

import jax
import jax.numpy as jnp
from jax import lax
from jax.experimental import pallas as pl
from jax.experimental.pallas import tpu as pltpu
from jax.experimental.pallas import tpu_sc as plsc

VOCAB = 1000000
EMBED = 32
OUT = 64
N = 16384 * 50
NC, NS = 2, 16
NW = NC * NS
B_PER_W = N // NW
CHUNK = 1280
NCHUNK = B_PER_W // CHUNK

BLK = 4096


def _make_gather():
    mesh = plsc.VectorSubcoreMesh(
        core_axis_name="c", subcore_axis_name="s", num_cores=NC, num_subcores=NS
    )

    @functools.partial(
        pl.kernel,
        out_type=jax.ShapeDtypeStruct((N, EMBED), jnp.float32),
        mesh=mesh,
        scratch_types=[
            pltpu.VMEM((CHUNK,), jnp.int32),
            pltpu.VMEM((CHUNK, EMBED), jnp.float32),
            pltpu.SemaphoreType.DMA,
        ],
    )
    def gather(idx_hbm, table_hbm, out_hbm, idx_v, rows_v, sem):
        wid = lax.axis_index("s") * NC + lax.axis_index("c")
        base = wid * B_PER_W

        def step(i, carry):
            off = base + i * CHUNK
            pltpu.sync_copy(idx_hbm.at[pl.ds(off, CHUNK)], idx_v)
            pltpu.async_copy(table_hbm.at[idx_v], rows_v, sem).wait()
            pltpu.sync_copy(rows_v, out_hbm.at[pl.ds(off, CHUNK)])
            return carry

        lax.fori_loop(0, NCHUNK, step, 0)

    return gather


_gather = _make_gather()


def _ln_body(emb_ref, w_ref, b_ref, g_ref, bt_ref, out_ref):
    emb = emb_ref[...]
    h = jnp.dot(emb, w_ref[...], preferred_element_type=jnp.float32) + b_ref[...]
    mu = jnp.mean(h, axis=-1, keepdims=True)
    var = jnp.mean(jnp.square(h - mu), axis=-1, keepdims=True)
    hn = (h - mu) * lax.rsqrt(var + 1e-5)
    out_ref[...] = hn * g_ref[...] + bt_ref[...]


def _linear_layernorm(emb, W, b, gamma, beta):
    b2 = b.reshape(1, OUT)
    g2 = gamma.reshape(1, OUT)
    bt2 = beta.reshape(1, OUT)
    return pl.pallas_call(
        _ln_body,
        grid=(N // BLK,),
        in_specs=[
            pl.BlockSpec((BLK, EMBED), lambda i: (i, 0)),
            pl.BlockSpec((EMBED, OUT), lambda i: (0, 0)),
            pl.BlockSpec((1, OUT), lambda i: (0, 0)),
            pl.BlockSpec((1, OUT), lambda i: (0, 0)),
            pl.BlockSpec((1, OUT), lambda i: (0, 0)),
        ],
        out_specs=pl.BlockSpec((BLK, OUT), lambda i: (i, 0)),
        out_shape=jax.ShapeDtypeStruct((N, OUT), jnp.float32),
    )(emb, W, b2, g2, bt2)


@jax.jit
def kernel(texts, table, W, b, gamma, beta):
    Bsz, T = texts.shape
    flat_ids = texts.reshape(-1).astype(jnp.int32)
    emb = _gather(flat_ids, table)
    out = _linear_layernorm(emb, W, b, gamma, beta)
    return out.reshape(Bsz, T, OUT)


# speedup vs baseline: 11.4989x; 11.4989x over previous
import functools

import jax
import jax.numpy as jnp
from jax import lax
from jax.experimental import pallas as pl
from jax.experimental.pallas import tpu as pltpu
from jax.experimental.pallas import tpu_sc as plsc

VOCAB = 1000000
EMBED = 32
OUT = 64
N = 16384 * 50
NC, NS = 2, 16
NW = NC * NS
B_PER_W = N // NW
CHUNK = 1280
NCHUNK = B_PER_W // CHUNK

BLK = 4096


@functools.lru_cache(maxsize=1)
def _make_gather():
    mesh = plsc.VectorSubcoreMesh(
        core_axis_name="c", subcore_axis_name="s", num_cores=NC, num_subcores=NS
    )

    @functools.partial(
        pl.kernel,
        out_type=jax.ShapeDtypeStruct((N, EMBED), jnp.float32),
        mesh=mesh,
        scratch_types=[
            pltpu.VMEM((CHUNK,), jnp.int32),
            pltpu.VMEM((CHUNK, EMBED), jnp.float32),
            pltpu.SemaphoreType.DMA,
        ],
        compiler_params=pltpu.CompilerParams(use_tc_tiling_on_sc=False),
    )
    def gather(idx_hbm, table_hbm, out_hbm, idx_v, rows_v, sem):
        wid = lax.axis_index("s") * NC + lax.axis_index("c")
        base = wid * B_PER_W

        def step(i, carry):
            off = base + i * CHUNK
            pltpu.sync_copy(idx_hbm.at[pl.ds(off, CHUNK)], idx_v)
            pltpu.async_copy(table_hbm.at[idx_v], rows_v, sem).wait()
            pltpu.sync_copy(rows_v, out_hbm.at[pl.ds(off, CHUNK)])
            return carry

        lax.fori_loop(0, NCHUNK, step, 0)

    return gather


def _ln_body(emb_ref, w_ref, b_ref, g_ref, bt_ref, out_ref):
    emb = emb_ref[...]
    h = jnp.dot(emb, w_ref[...], preferred_element_type=jnp.float32) + b_ref[...]
    mu = jnp.mean(h, axis=-1, keepdims=True)
    var = jnp.mean(jnp.square(h - mu), axis=-1, keepdims=True)
    hn = (h - mu) * lax.rsqrt(var + 1e-5)
    out_ref[...] = hn * g_ref[...] + bt_ref[...]


def _linear_layernorm(emb, W, b, gamma, beta):
    b2 = b.reshape(1, OUT)
    g2 = gamma.reshape(1, OUT)
    bt2 = beta.reshape(1, OUT)
    return pl.pallas_call(
        _ln_body,
        grid=(N // BLK,),
        in_specs=[
            pl.BlockSpec((BLK, EMBED), lambda i: (i, 0)),
            pl.BlockSpec((EMBED, OUT), lambda i: (0, 0)),
            pl.BlockSpec((1, OUT), lambda i: (0, 0)),
            pl.BlockSpec((1, OUT), lambda i: (0, 0)),
            pl.BlockSpec((1, OUT), lambda i: (0, 0)),
        ],
        out_specs=pl.BlockSpec((BLK, OUT), lambda i: (i, 0)),
        out_shape=jax.ShapeDtypeStruct((N, OUT), jnp.float32),
    )(emb, W, b2, g2, bt2)


@jax.jit
def kernel(texts, table, W, b, gamma, beta):
    Bsz, T = texts.shape
    flat_ids = texts.reshape(-1).astype(jnp.int32)
    emb = _make_gather()(flat_ids, table)
    out = _linear_layernorm(emb, W, b, gamma, beta)
    return out.reshape(Bsz, T, OUT)


# speedup vs baseline: 17.3725x; 1.5108x over previous
import functools

import jax
import jax.numpy as jnp
from jax import lax
from jax.experimental import pallas as pl
from jax.experimental.pallas import tpu as pltpu
from jax.experimental.pallas import tpu_sc as plsc

VOCAB = 1000000
EMBED = 32
OUT = 64
N = 16384 * 50
NC, NS = 2, 16
NW = NC * NS
B_PER_W = N // NW
CHUNK = 1280
NCHUNK = B_PER_W // CHUNK

BLK = 4096


@functools.lru_cache(maxsize=1)
def _make_gather():
    mesh = plsc.VectorSubcoreMesh(
        core_axis_name="c", subcore_axis_name="s", num_cores=NC, num_subcores=NS
    )

    @functools.partial(
        pl.kernel,
        out_type=jax.ShapeDtypeStruct((N, EMBED), jnp.float32),
        mesh=mesh,
        scratch_types=[
            pltpu.VMEM((CHUNK,), jnp.int32),
            pltpu.VMEM((CHUNK, EMBED), jnp.float32),
            pltpu.SemaphoreType.DMA,
        ],
        compiler_params=pltpu.CompilerParams(use_tc_tiling_on_sc=False),
    )
    def gather(idx_hbm, table_hbm, out_hbm, idx_v, rows_v, sem):
        wid = lax.axis_index("s") * NC + lax.axis_index("c")
        base = wid * B_PER_W

        def step(i, carry):
            off = base + i * CHUNK
            pltpu.sync_copy(idx_hbm.at[pl.ds(off, CHUNK)], idx_v)
            pltpu.async_copy(table_hbm.at[idx_v], rows_v, sem).wait()
            pltpu.sync_copy(rows_v, out_hbm.at[pl.ds(off, CHUNK)])
            return carry

        lax.fori_loop(0, NCHUNK, step, 0)

    return gather


PACK = 128 // EMBED
NR = N // PACK
LANES_IN = PACK * EMBED
LANES_OUT = PACK * OUT


def _ln_body(x_ref, wcat_ref, bcat_ref, a_ref, g_ref, bt_ref, out_ref):
    x = x_ref[...]
    hh = (
        jnp.dot(x, wcat_ref[...], preferred_element_type=jnp.float32)
        + bcat_ref[...]
    )
    h = hh[:, :LANES_OUT]
    mu = hh[:, LANES_OUT:]
    d = h - mu
    var = jnp.dot(d * d, a_ref[...], preferred_element_type=jnp.float32)
    out_ref[...] = d * lax.rsqrt(var + 1e-5) * g_ref[...] + bt_ref[...]


def _linear_layernorm(emb128, W, b, gamma, beta):
    eye = jnp.eye(PACK, dtype=jnp.float32)
    wbig = jnp.kron(eye, W)
    avg = jnp.kron(eye, jnp.full((OUT, OUT), 1.0 / OUT, jnp.float32))
    wcat = jnp.concatenate([wbig, wbig @ avg], axis=1)
    b4 = jnp.tile(b, PACK).reshape(1, LANES_OUT)
    bcat = jnp.concatenate([b4, b4 @ avg], axis=1)
    g4 = jnp.tile(gamma, PACK).reshape(1, LANES_OUT)
    bt4 = jnp.tile(beta, PACK).reshape(1, LANES_OUT)
    return pl.pallas_call(
        _ln_body,
        grid=(NR // BLK,),
        in_specs=[
            pl.BlockSpec((BLK, LANES_IN), lambda i: (i, 0)),
            pl.BlockSpec((LANES_IN, 2 * LANES_OUT), lambda i: (0, 0)),
            pl.BlockSpec((1, 2 * LANES_OUT), lambda i: (0, 0)),
            pl.BlockSpec((LANES_OUT, LANES_OUT), lambda i: (0, 0)),
            pl.BlockSpec((1, LANES_OUT), lambda i: (0, 0)),
            pl.BlockSpec((1, LANES_OUT), lambda i: (0, 0)),
        ],
        out_specs=pl.BlockSpec((BLK, LANES_OUT), lambda i: (i, 0)),
        out_shape=jax.ShapeDtypeStruct((NR, LANES_OUT), jnp.float32),
    )(emb128, wcat, bcat, avg, g4, bt4)


@jax.jit
def kernel(texts, table, W, b, gamma, beta):
    Bsz, T = texts.shape
    flat_ids = texts.reshape(-1).astype(jnp.int32)
    emb = _make_gather()(flat_ids, table)
    emb128 = emb.reshape(NR, LANES_IN)
    out = _linear_layernorm(emb128, W, b, gamma, beta)
    return out.reshape(Bsz, T, OUT)
